# baseline (speedup 1.0000x reference)
import functools

import jax
import jax.numpy as jnp
from jax import lax
from jax.experimental import pallas as pl
from jax.experimental.pallas import tpu as pltpu
from jax.experimental.pallas import tpu_sc as plsc

TOPK = 2
BLK = 256
RBLK = 512
NC, NS, LANES = 2, 16, 16
NW = NC * NS

_SC_MESH = dict(core_axis_name="c", subcore_axis_name="s",
                num_cores=NC, num_subcores=NS)


def _router_body(x_ref, gw_ref, logits_ref, topw_ref, topi_ref, xpk_ref):
    x = x_ref[...]
    logits = lax.dot_general(x, gw_ref[...], (((1,), (1,)), ((), ())),
                             preferred_element_type=jnp.float32)
    logits_ref[...] = logits
    d2 = x.shape[1] // 2
    lo = x[:, :d2].astype(jnp.bfloat16).astype(jnp.float32)
    hi = x[:, d2:].astype(jnp.bfloat16).astype(jnp.float32)
    lo_u = lax.bitcast_convert_type(lo, jnp.uint32) >> 16
    hi_u = lax.bitcast_convert_type(hi, jnp.uint32) & jnp.uint32(0xFFFF0000)
    xpk_ref[...] = lax.bitcast_convert_type(hi_u | lo_u, jnp.int32)
    e = logits.shape[1]
    m = jnp.max(logits, axis=1, keepdims=True)
    p = jnp.exp(logits - m)
    probs = p / jnp.sum(p, axis=1, keepdims=True)
    iota = lax.broadcasted_iota(jnp.int32, probs.shape, 1)
    m1 = jnp.max(probs, axis=1, keepdims=True)
    i1 = jnp.min(jnp.where(probs == m1, iota, e), axis=1, keepdims=True)
    probs2 = jnp.where(iota == i1, -jnp.inf, probs)
    m2 = jnp.max(probs2, axis=1, keepdims=True)
    i2 = jnp.min(jnp.where(probs2 == m2, iota, e), axis=1, keepdims=True)
    topw_ref[...] = jnp.concatenate([m1, m2], axis=1)
    topi_ref[...] = jnp.concatenate([i1, i2], axis=1)


def _mlp_body(be_ref, bv_ref, xs_ref, wg_ref, wu_ref, wd_ref, w_ref, out_ref):
    i = pl.program_id(0)

    @pl.when(bv_ref[i] != 0)
    def _():
        xi = lax.bitcast_convert_type(xs_ref[...], jnp.uint32)
        x_lo = lax.bitcast_convert_type(xi << 16,
                                        jnp.float32).astype(jnp.bfloat16)
        x_hi = lax.bitcast_convert_type(xi & jnp.uint32(0xFFFF0000),
                                        jnp.float32).astype(jnp.bfloat16)
        d2 = xi.shape[1]
        dn = (((1,), (1,)), ((), ()))
        wg, wu, wd = wg_ref[0], wu_ref[0], wd_ref[0]
        g = (lax.dot_general(x_lo, wg[:, :d2], dn,
                             preferred_element_type=jnp.float32)
             + lax.dot_general(x_hi, wg[:, d2:], dn,
                               preferred_element_type=jnp.float32))
        u = (lax.dot_general(x_lo, wu[:, :d2], dn,
                             preferred_element_type=jnp.float32)
             + lax.dot_general(x_hi, wu[:, d2:], dn,
                               preferred_element_type=jnp.float32))
        hv = ((g * jax.nn.sigmoid(g)) * u).astype(jnp.bfloat16)
        o = lax.dot_general(hv, wd, dn, preferred_element_type=jnp.float32)
        out_ref[...] = o * w_ref[...]


def _sc_gather(x, row_token, npad):
    t, d = x.shape
    b_per_w = npad // NW
    ch = 40
    n = b_per_w // ch
    mesh = plsc.VectorSubcoreMesh(**_SC_MESH)

    @functools.partial(
        pl.kernel, mesh=mesh,
        out_type=jax.ShapeDtypeStruct((npad, d), jnp.int32),
        scratch_types=[pltpu.VMEM((ch,), jnp.int32),
                       pltpu.VMEM((ch,), jnp.int32),
                       pltpu.VMEM((ch, d), jnp.int32),
                       pltpu.VMEM((ch, d), jnp.int32),
                       pltpu.SemaphoreType.DMA,
                       pltpu.SemaphoreType.DMA,
                       pltpu.SemaphoreType.DMA,
                       pltpu.SemaphoreType.DMA],
    )
    def gather_k(x_hbm, tok_hbm, xs_hbm, idx0, idx1, rows0, rows1,
                 g0, g1, w0, w1):
        idx, rows, gs, ws = [idx0, idx1], [rows0, rows1], [g0, g1], [w0, w1]
        wid = lax.axis_index("s") * NC + lax.axis_index("c")
        base = wid * b_per_w
        gd, wd = [None] * n, [None] * n
        pltpu.sync_copy(tok_hbm.at[pl.ds(base, ch)], idx[0])
        gd[0] = pltpu.async_copy(x_hbm.at[idx[0]], rows[0], gs[0])
        for c in range(n):
            cur = c & 1
            nxt = 1 - cur
            if c + 1 < n:
                pltpu.sync_copy(tok_hbm.at[pl.ds(base + (c + 1) * ch, ch)],
                                idx[nxt])
                if c >= 1:
                    wd[c - 1].wait()
                gd[c + 1] = pltpu.async_copy(x_hbm.at[idx[nxt]], rows[nxt],
                                             gs[nxt])
            gd[c].wait()
            wd[c] = pltpu.async_copy(rows[cur],
                                     xs_hbm.at[pl.ds(base + c * ch, ch)],
                                     ws[cur])
        if n >= 2:
            wd[n - 2].wait()
        wd[n - 1].wait()

    return gather_k(x, row_token)


def _sc_combine(wout, pos0, pos1):
    t = pos0.shape[0]
    d = wout.shape[1]
    t_per_w = t // NW
    ch = 8
    n = t_per_w // ch
    mesh = plsc.VectorSubcoreMesh(**_SC_MESH)

    @functools.partial(
        pl.kernel, mesh=mesh,
        out_type=jax.ShapeDtypeStruct((t, d), jnp.float32),
        scratch_types=[pltpu.VMEM((ch,), jnp.int32),
                       pltpu.VMEM((ch,), jnp.int32),
                       pltpu.VMEM((ch,), jnp.int32),
                       pltpu.VMEM((ch,), jnp.int32),
                       pltpu.VMEM((ch, d), jnp.float32),
                       pltpu.VMEM((ch, d), jnp.float32),
                       pltpu.VMEM((ch, d), jnp.float32),
                       pltpu.VMEM((ch, d), jnp.float32),
                       pltpu.SemaphoreType.DMA,
                       pltpu.SemaphoreType.DMA,
                       pltpu.SemaphoreType.DMA,
                       pltpu.SemaphoreType.DMA],
    )
    def combine_k(wout_hbm, p0_hbm, p1_hbm, out_hbm,
                  p0a, p0b, p1a, p1b, r0a, r0b, r1a, r1b, g0, g1, w0, w1):
        p0, p1 = [p0a, p0b], [p1a, p1b]
        r0, r1, gs, ws = [r0a, r0b], [r1a, r1b], [g0, g1], [w0, w1]
        wid = lax.axis_index("s") * NC + lax.axis_index("c")
        base = wid * t_per_w
        g0d, g1d, wd = [None] * n, [None] * n, [None] * n
        pltpu.sync_copy(p0_hbm.at[pl.ds(base, ch)], p0[0])
        pltpu.sync_copy(p1_hbm.at[pl.ds(base, ch)], p1[0])
        g0d[0] = pltpu.async_copy(wout_hbm.at[p0[0]], r0[0], gs[0])
        g1d[0] = pltpu.async_copy(wout_hbm.at[p1[0]], r1[0], gs[0])
        for c in range(n):
            cur = c & 1
            nxt = 1 - cur
            if c + 1 < n:
                off_n = base + (c + 1) * ch
                pltpu.sync_copy(p0_hbm.at[pl.ds(off_n, ch)], p0[nxt])
                pltpu.sync_copy(p1_hbm.at[pl.ds(off_n, ch)], p1[nxt])
                if c >= 1:
                    wd[c - 1].wait()
                g0d[c + 1] = pltpu.async_copy(wout_hbm.at[p0[nxt]], r0[nxt],
                                              gs[nxt])
                g1d[c + 1] = pltpu.async_copy(wout_hbm.at[p1[nxt]], r1[nxt],
                                              gs[nxt])
            g0d[c].wait()
            g1d[c].wait()
            for r in range(ch):
                def add_body(ci, _, r=r, cur=cur):
                    sl = pl.ds(ci * LANES, LANES)
                    r0[cur][r, sl] = r0[cur][r, sl] + r1[cur][r, sl]
                    return 0
                lax.fori_loop(0, d // LANES, add_body, 0)
            wd[c] = pltpu.async_copy(r0[cur],
                                     out_hbm.at[pl.ds(base + c * ch, ch)],
                                     ws[cur])
        if n >= 2:
            wd[n - 2].wait()
        wd[n - 1].wait()

    return combine_k(wout, pos0, pos1)


def kernel(hidden_states, gate_w, Wg, Wu, Wd):
    b, s, d = hidden_states.shape
    e, h, _ = Wg.shape
    t = b * s
    p = t * TOPK
    nb = (p + e * (BLK - 1) + BLK - 1) // BLK
    npad = nb * BLK

    x = hidden_states.reshape(t, d)

    logits, topw, topi, x_pk = pl.pallas_call(
        _router_body,
        grid=(t // RBLK,),
        in_specs=[pl.BlockSpec((RBLK, d), lambda i: (i, 0)),
                  pl.BlockSpec((e, d), lambda i: (0, 0))],
        out_specs=[pl.BlockSpec((RBLK, e), lambda i: (i, 0)),
                   pl.BlockSpec((RBLK, TOPK), lambda i: (i, 0)),
                   pl.BlockSpec((RBLK, TOPK), lambda i: (i, 0)),
                   pl.BlockSpec((RBLK, d // 2), lambda i: (i, 0))],
        out_shape=[jax.ShapeDtypeStruct((t, e), jnp.float32),
                   jax.ShapeDtypeStruct((t, TOPK), jnp.float32),
                   jax.ShapeDtypeStruct((t, TOPK), jnp.int32),
                   jax.ShapeDtypeStruct((t, d // 2), jnp.int32)],
    )(x, gate_w)

    pair_e = topi.reshape(-1)
    pair_w = topw.reshape(-1)
    sort_idx = jnp.argsort(pair_e, stable=True)
    se = pair_e[sort_idx]
    counts = jnp.zeros((e,), jnp.int32).at[pair_e].add(1)
    pad_counts = ((counts + BLK - 1) // BLK) * BLK
    ends = jnp.cumsum(pad_counts)
    pad_off = ends - pad_counts
    un_off = jnp.cumsum(counts) - counts
    rank = jnp.arange(p, dtype=jnp.int32) - un_off[se]
    dest = pad_off[se] + rank
    row_token = jnp.zeros((npad,), jnp.int32).at[dest].set(
        (sort_idx // TOPK).astype(jnp.int32))
    row_w = jnp.zeros((npad,), jnp.float32).at[dest].set(pair_w[sort_idx])
    pos = jnp.zeros((p,), jnp.int32).at[sort_idx].set(dest)
    pos0 = pos[0::TOPK]
    pos1 = pos[1::TOPK]
    total = ends[-1]
    bstart = jnp.arange(nb, dtype=jnp.int32) * BLK
    block_expert = jnp.minimum(
        jnp.searchsorted(ends, bstart, side="right").astype(jnp.int32), e - 1)
    block_valid = (bstart < total).astype(jnp.int32)

    xs = _sc_gather(x_pk, row_token, npad)

    grid_spec = pltpu.PrefetchScalarGridSpec(
        num_scalar_prefetch=2,
        grid=(nb,),
        in_specs=[
            pl.BlockSpec((BLK, d // 2), lambda i, be, bv: (i, 0)),
            pl.BlockSpec((1, h, d), lambda i, be, bv: (be[i], 0, 0)),
            pl.BlockSpec((1, h, d), lambda i, be, bv: (be[i], 0, 0)),
            pl.BlockSpec((1, d, h), lambda i, be, bv: (be[i], 0, 0)),
            pl.BlockSpec((BLK, 1), lambda i, be, bv: (i, 0)),
        ],
        out_specs=pl.BlockSpec((BLK, d), lambda i, be, bv: (i, 0)),
    )
    wout = pl.pallas_call(
        _mlp_body,
        grid_spec=grid_spec,
        out_shape=jax.ShapeDtypeStruct((npad, d), jnp.float32),
    )(block_expert, block_valid, xs, Wg.astype(jnp.bfloat16),
      Wu.astype(jnp.bfloat16), Wd.astype(jnp.bfloat16),
      row_w.reshape(npad, 1))

    final = _sc_combine(wout, pos0, pos1)

    return final.reshape(b, s, d), logits


# speedup vs baseline: 1.3119x; 1.3119x over previous
import functools

import jax
import jax.numpy as jnp
from jax import lax
from jax.experimental import pallas as pl
from jax.experimental.pallas import tpu as pltpu
from jax.experimental.pallas import tpu_sc as plsc

TOPK = 2
BLK = 256
RBLK = 512
NC, NS, LANES = 2, 16, 16
NW = NC * NS

_SC_MESH = dict(core_axis_name="c", subcore_axis_name="s",
                num_cores=NC, num_subcores=NS)


def _router_body(x_ref, gw_ref, logits_ref, topw_ref, topi_ref, xpk_ref):
    x = x_ref[...]
    logits = lax.dot_general(x, gw_ref[...], (((1,), (1,)), ((), ())),
                             preferred_element_type=jnp.float32)
    logits_ref[...] = logits
    d2 = x.shape[1] // 2
    lo = x[:, :d2].astype(jnp.bfloat16).astype(jnp.float32)
    hi = x[:, d2:].astype(jnp.bfloat16).astype(jnp.float32)
    lo_u = lax.bitcast_convert_type(lo, jnp.uint32) >> 16
    hi_u = lax.bitcast_convert_type(hi, jnp.uint32) & jnp.uint32(0xFFFF0000)
    xpk_ref[...] = lax.bitcast_convert_type(hi_u | lo_u, jnp.int32)
    e = logits.shape[1]
    m = jnp.max(logits, axis=1, keepdims=True)
    p = jnp.exp(logits - m)
    probs = p / jnp.sum(p, axis=1, keepdims=True)
    iota = lax.broadcasted_iota(jnp.int32, probs.shape, 1)
    m1 = jnp.max(probs, axis=1, keepdims=True)
    i1 = jnp.min(jnp.where(probs == m1, iota, e), axis=1, keepdims=True)
    probs2 = jnp.where(iota == i1, -jnp.inf, probs)
    m2 = jnp.max(probs2, axis=1, keepdims=True)
    i2 = jnp.min(jnp.where(probs2 == m2, iota, e), axis=1, keepdims=True)
    topw_ref[...] = jnp.concatenate([m1, m2], axis=1)
    topi_ref[...] = jnp.concatenate([i1, i2], axis=1)


def _mlp_body(be_ref, bv_ref, xs_ref, wg_ref, wu_ref, wd_ref, w_ref, out_ref):
    i = pl.program_id(0)

    @pl.when(bv_ref[i] != 0)
    def _():
        xi = lax.bitcast_convert_type(xs_ref[...], jnp.uint32)
        x_lo = lax.bitcast_convert_type(xi << 16, jnp.float32)
        x_hi = lax.bitcast_convert_type(xi & jnp.uint32(0xFFFF0000),
                                        jnp.float32)
        d2 = xi.shape[1]
        dn = (((1,), (1,)), ((), ()))
        wg, wu, wd = wg_ref[0], wu_ref[0], wd_ref[0]
        g = (lax.dot_general(x_lo, wg[:, :d2], dn,
                             preferred_element_type=jnp.float32)
             + lax.dot_general(x_hi, wg[:, d2:], dn,
                               preferred_element_type=jnp.float32))
        u = (lax.dot_general(x_lo, wu[:, :d2], dn,
                             preferred_element_type=jnp.float32)
             + lax.dot_general(x_hi, wu[:, d2:], dn,
                               preferred_element_type=jnp.float32))
        hv = (g * jax.nn.sigmoid(g)) * u
        o = lax.dot_general(hv, wd, dn, preferred_element_type=jnp.float32)
        out_ref[...] = o * w_ref[...]


def _sc_gather(x, row_token, npad):
    t, d = x.shape
    b_per_w = npad // NW
    ch = 40
    n = b_per_w // ch
    mesh = plsc.VectorSubcoreMesh(**_SC_MESH)

    @functools.partial(
        pl.kernel, mesh=mesh,
        out_type=jax.ShapeDtypeStruct((npad, d), jnp.int32),
        scratch_types=[pltpu.VMEM((ch,), jnp.int32),
                       pltpu.VMEM((ch,), jnp.int32),
                       pltpu.VMEM((ch, d), jnp.int32),
                       pltpu.VMEM((ch, d), jnp.int32),
                       pltpu.SemaphoreType.DMA,
                       pltpu.SemaphoreType.DMA,
                       pltpu.SemaphoreType.DMA,
                       pltpu.SemaphoreType.DMA],
    )
    def gather_k(x_hbm, tok_hbm, xs_hbm, idx0, idx1, rows0, rows1,
                 g0, g1, w0, w1):
        idx, rows, gs, ws = [idx0, idx1], [rows0, rows1], [g0, g1], [w0, w1]
        wid = lax.axis_index("s") * NC + lax.axis_index("c")
        base = wid * b_per_w
        gd, wd = [None] * n, [None] * n
        pltpu.sync_copy(tok_hbm.at[pl.ds(base, ch)], idx[0])
        gd[0] = pltpu.async_copy(x_hbm.at[idx[0]], rows[0], gs[0])
        for c in range(n):
            cur = c & 1
            nxt = 1 - cur
            if c + 1 < n:
                pltpu.sync_copy(tok_hbm.at[pl.ds(base + (c + 1) * ch, ch)],
                                idx[nxt])
                if c >= 1:
                    wd[c - 1].wait()
                gd[c + 1] = pltpu.async_copy(x_hbm.at[idx[nxt]], rows[nxt],
                                             gs[nxt])
            gd[c].wait()
            wd[c] = pltpu.async_copy(rows[cur],
                                     xs_hbm.at[pl.ds(base + c * ch, ch)],
                                     ws[cur])
        if n >= 2:
            wd[n - 2].wait()
        wd[n - 1].wait()

    return gather_k(x, row_token)


def _sc_combine(wout, pos0, pos1):
    t = pos0.shape[0]
    d = wout.shape[1]
    t_per_w = t // NW
    ch = 8
    n = t_per_w // ch
    mesh = plsc.VectorSubcoreMesh(**_SC_MESH)

    @functools.partial(
        pl.kernel, mesh=mesh,
        out_type=jax.ShapeDtypeStruct((t, d), jnp.float32),
        scratch_types=[pltpu.VMEM((ch,), jnp.int32),
                       pltpu.VMEM((ch,), jnp.int32),
                       pltpu.VMEM((ch,), jnp.int32),
                       pltpu.VMEM((ch,), jnp.int32),
                       pltpu.VMEM((ch, d), jnp.float32),
                       pltpu.VMEM((ch, d), jnp.float32),
                       pltpu.VMEM((ch, d), jnp.float32),
                       pltpu.VMEM((ch, d), jnp.float32),
                       pltpu.SemaphoreType.DMA,
                       pltpu.SemaphoreType.DMA,
                       pltpu.SemaphoreType.DMA,
                       pltpu.SemaphoreType.DMA],
    )
    def combine_k(wout_hbm, p0_hbm, p1_hbm, out_hbm,
                  p0a, p0b, p1a, p1b, r0a, r0b, r1a, r1b, g0, g1, w0, w1):
        p0, p1 = [p0a, p0b], [p1a, p1b]
        r0, r1, gs, ws = [r0a, r0b], [r1a, r1b], [g0, g1], [w0, w1]
        wid = lax.axis_index("s") * NC + lax.axis_index("c")
        base = wid * t_per_w
        g0d, g1d, wd = [None] * n, [None] * n, [None] * n
        pltpu.sync_copy(p0_hbm.at[pl.ds(base, ch)], p0[0])
        pltpu.sync_copy(p1_hbm.at[pl.ds(base, ch)], p1[0])
        g0d[0] = pltpu.async_copy(wout_hbm.at[p0[0]], r0[0], gs[0])
        g1d[0] = pltpu.async_copy(wout_hbm.at[p1[0]], r1[0], gs[0])
        for c in range(n):
            cur = c & 1
            nxt = 1 - cur
            if c + 1 < n:
                off_n = base + (c + 1) * ch
                pltpu.sync_copy(p0_hbm.at[pl.ds(off_n, ch)], p0[nxt])
                pltpu.sync_copy(p1_hbm.at[pl.ds(off_n, ch)], p1[nxt])
                if c >= 1:
                    wd[c - 1].wait()
                g0d[c + 1] = pltpu.async_copy(wout_hbm.at[p0[nxt]], r0[nxt],
                                              gs[nxt])
                g1d[c + 1] = pltpu.async_copy(wout_hbm.at[p1[nxt]], r1[nxt],
                                              gs[nxt])
            g0d[c].wait()
            g1d[c].wait()
            for r in range(ch):
                def add_body(ci, _, r=r, cur=cur):
                    sl = pl.ds(ci * LANES, LANES)
                    r0[cur][r, sl] = r0[cur][r, sl] + r1[cur][r, sl]
                    return 0
                lax.fori_loop(0, d // LANES, add_body, 0)
            wd[c] = pltpu.async_copy(r0[cur],
                                     out_hbm.at[pl.ds(base + c * ch, ch)],
                                     ws[cur])
        if n >= 2:
            wd[n - 2].wait()
        wd[n - 1].wait()

    return combine_k(wout, pos0, pos1)


def kernel(hidden_states, gate_w, Wg, Wu, Wd):
    b, s, d = hidden_states.shape
    e, h, _ = Wg.shape
    t = b * s
    p = t * TOPK
    nb = (p + e * (BLK - 1) + BLK - 1) // BLK
    npad = nb * BLK

    x = hidden_states.reshape(t, d)

    logits, topw, topi, x_pk = pl.pallas_call(
        _router_body,
        grid=(t // RBLK,),
        in_specs=[pl.BlockSpec((RBLK, d), lambda i: (i, 0)),
                  pl.BlockSpec((e, d), lambda i: (0, 0))],
        out_specs=[pl.BlockSpec((RBLK, e), lambda i: (i, 0)),
                   pl.BlockSpec((RBLK, TOPK), lambda i: (i, 0)),
                   pl.BlockSpec((RBLK, TOPK), lambda i: (i, 0)),
                   pl.BlockSpec((RBLK, d // 2), lambda i: (i, 0))],
        out_shape=[jax.ShapeDtypeStruct((t, e), jnp.float32),
                   jax.ShapeDtypeStruct((t, TOPK), jnp.float32),
                   jax.ShapeDtypeStruct((t, TOPK), jnp.int32),
                   jax.ShapeDtypeStruct((t, d // 2), jnp.int32)],
    )(x, gate_w)

    pair_e = topi.reshape(-1)
    pair_w = topw.reshape(-1)
    onehot = (pair_e[:, None] ==
              jnp.arange(e, dtype=jnp.int32)[None, :]).astype(jnp.int32)
    csum = jnp.cumsum(onehot, axis=0)
    counts = csum[-1]
    pad_counts = ((counts + BLK - 1) // BLK) * BLK
    ends = jnp.cumsum(pad_counts)
    pad_off = ends - pad_counts
    rank = jnp.take_along_axis(csum, pair_e[:, None], axis=1)[:, 0] - 1
    dest = pad_off[pair_e] + rank
    row_token = jnp.zeros((npad,), jnp.int32).at[dest].set(
        jnp.arange(p, dtype=jnp.int32) // TOPK)
    row_w = jnp.zeros((npad,), jnp.float32).at[dest].set(pair_w)
    pos0 = dest[0::TOPK]
    pos1 = dest[1::TOPK]
    total = ends[-1]
    bstart = jnp.arange(nb, dtype=jnp.int32) * BLK
    block_expert = jnp.minimum(
        jnp.searchsorted(ends, bstart, side="right").astype(jnp.int32), e - 1)
    block_valid = (bstart < total).astype(jnp.int32)

    xs = _sc_gather(x_pk, row_token, npad)

    grid_spec = pltpu.PrefetchScalarGridSpec(
        num_scalar_prefetch=2,
        grid=(nb,),
        in_specs=[
            pl.BlockSpec((BLK, d // 2), lambda i, be, bv: (i, 0)),
            pl.BlockSpec((1, h, d), lambda i, be, bv: (be[i], 0, 0)),
            pl.BlockSpec((1, h, d), lambda i, be, bv: (be[i], 0, 0)),
            pl.BlockSpec((1, d, h), lambda i, be, bv: (be[i], 0, 0)),
            pl.BlockSpec((BLK, 1), lambda i, be, bv: (i, 0)),
        ],
        out_specs=pl.BlockSpec((BLK, d), lambda i, be, bv: (i, 0)),
    )
    wout = pl.pallas_call(
        _mlp_body,
        grid_spec=grid_spec,
        out_shape=jax.ShapeDtypeStruct((npad, d), jnp.float32),
    )(block_expert, block_valid, xs, Wg, Wu, Wd, row_w.reshape(npad, 1))

    final = _sc_combine(wout, pos0, pos1)

    return final.reshape(b, s, d), logits


# speedup vs baseline: 1.3160x; 1.0031x over previous
import functools

import jax
import jax.numpy as jnp
from jax import lax
from jax.experimental import pallas as pl
from jax.experimental.pallas import tpu as pltpu
from jax.experimental.pallas import tpu_sc as plsc

TOPK = 2
BLK = 256
RBLK = 512
NC, NS, LANES = 2, 16, 16
NW = NC * NS

_SC_MESH = dict(core_axis_name="c", subcore_axis_name="s",
                num_cores=NC, num_subcores=NS)


def _router_body(x_ref, gw_ref, logits_ref, topw_ref, topi_ref, xpk_ref):
    x = x_ref[...]
    logits = lax.dot_general(x, gw_ref[...], (((1,), (1,)), ((), ())),
                             preferred_element_type=jnp.float32)
    logits_ref[...] = logits
    d2 = x.shape[1] // 2
    lo = x[:, :d2].astype(jnp.bfloat16).astype(jnp.float32)
    hi = x[:, d2:].astype(jnp.bfloat16).astype(jnp.float32)
    lo_u = lax.bitcast_convert_type(lo, jnp.uint32) >> 16
    hi_u = lax.bitcast_convert_type(hi, jnp.uint32) & jnp.uint32(0xFFFF0000)
    xpk_ref[...] = lax.bitcast_convert_type(hi_u | lo_u, jnp.int32)
    e = logits.shape[1]
    m = jnp.max(logits, axis=1, keepdims=True)
    p = jnp.exp(logits - m)
    probs = p / jnp.sum(p, axis=1, keepdims=True)
    iota = lax.broadcasted_iota(jnp.int32, probs.shape, 1)
    m1 = jnp.max(probs, axis=1, keepdims=True)
    i1 = jnp.min(jnp.where(probs == m1, iota, e), axis=1, keepdims=True)
    probs2 = jnp.where(iota == i1, -jnp.inf, probs)
    m2 = jnp.max(probs2, axis=1, keepdims=True)
    i2 = jnp.min(jnp.where(probs2 == m2, iota, e), axis=1, keepdims=True)
    topw_ref[...] = jnp.concatenate([m1, m2], axis=1)
    topi_ref[...] = jnp.concatenate([i1, i2], axis=1)


def _mlp_body(be_ref, bv_ref, xs_ref, wg_ref, wu_ref, wd_ref, w_ref, out_ref):
    _mlp_compute(be_ref, bv_ref, xs_ref, wg_ref, wu_ref, wd_ref, w_ref,
                 out_ref)


def _mlp_body_cont(be_ref, bv_ref, prev_ref, xs_ref, wg_ref, wu_ref, wd_ref,
                   w_ref, out_ref):
    del prev_ref
    _mlp_compute(be_ref, bv_ref, xs_ref, wg_ref, wu_ref, wd_ref, w_ref,
                 out_ref)


def _mlp_compute(be_ref, bv_ref, xs_ref, wg_ref, wu_ref, wd_ref, w_ref,
                 out_ref):
    i = pl.program_id(0)

    @pl.when(bv_ref[i] != 0)
    def _():
        xi = lax.bitcast_convert_type(xs_ref[...], jnp.uint32)
        x_lo = lax.bitcast_convert_type(xi << 16, jnp.float32)
        x_hi = lax.bitcast_convert_type(xi & jnp.uint32(0xFFFF0000),
                                        jnp.float32)
        d2 = xi.shape[1]
        dn = (((1,), (1,)), ((), ()))
        wg, wu, wd = wg_ref[0], wu_ref[0], wd_ref[0]
        g = (lax.dot_general(x_lo, wg[:, :d2], dn,
                             preferred_element_type=jnp.float32)
             + lax.dot_general(x_hi, wg[:, d2:], dn,
                               preferred_element_type=jnp.float32))
        u = (lax.dot_general(x_lo, wu[:, :d2], dn,
                             preferred_element_type=jnp.float32)
             + lax.dot_general(x_hi, wu[:, d2:], dn,
                               preferred_element_type=jnp.float32))
        hv = (g * jax.nn.sigmoid(g)) * u
        o = lax.dot_general(hv, wd, dn, preferred_element_type=jnp.float32)
        out_ref[...] = o * w_ref[...]


def _sc_gather(x, row_token, npad):
    t, d = x.shape
    b_per_w = npad // NW
    ch = 40
    n = b_per_w // ch
    mesh = plsc.VectorSubcoreMesh(**_SC_MESH)

    @functools.partial(
        pl.kernel, mesh=mesh,
        out_type=jax.ShapeDtypeStruct((npad, d), jnp.int32),
        scratch_types=[pltpu.VMEM((ch,), jnp.int32),
                       pltpu.VMEM((ch,), jnp.int32),
                       pltpu.VMEM((ch, d), jnp.int32),
                       pltpu.VMEM((ch, d), jnp.int32),
                       pltpu.SemaphoreType.DMA,
                       pltpu.SemaphoreType.DMA,
                       pltpu.SemaphoreType.DMA,
                       pltpu.SemaphoreType.DMA],
    )
    def gather_k(x_hbm, tok_hbm, xs_hbm, idx0, idx1, rows0, rows1,
                 g0, g1, w0, w1):
        idx, rows, gs, ws = [idx0, idx1], [rows0, rows1], [g0, g1], [w0, w1]
        wid = lax.axis_index("s") * NC + lax.axis_index("c")
        base = wid * b_per_w
        gd, wd = [None] * n, [None] * n
        pltpu.sync_copy(tok_hbm.at[pl.ds(base, ch)], idx[0])
        gd[0] = pltpu.async_copy(x_hbm.at[idx[0]], rows[0], gs[0])
        for c in range(n):
            cur = c & 1
            nxt = 1 - cur
            if c + 1 < n:
                pltpu.sync_copy(tok_hbm.at[pl.ds(base + (c + 1) * ch, ch)],
                                idx[nxt])
                if c >= 1:
                    wd[c - 1].wait()
                gd[c + 1] = pltpu.async_copy(x_hbm.at[idx[nxt]], rows[nxt],
                                             gs[nxt])
            gd[c].wait()
            wd[c] = pltpu.async_copy(rows[cur],
                                     xs_hbm.at[pl.ds(base + c * ch, ch)],
                                     ws[cur])
        if n >= 2:
            wd[n - 2].wait()
        wd[n - 1].wait()

    return gather_k(x, row_token)


def _sc_combine(wout, pos0, pos1):
    t = pos0.shape[0]
    d = wout.shape[1]
    t_per_w = t // NW
    ch = 8
    n = t_per_w // ch
    mesh = plsc.VectorSubcoreMesh(**_SC_MESH)

    @functools.partial(
        pl.kernel, mesh=mesh,
        out_type=jax.ShapeDtypeStruct((t, d), jnp.float32),
        scratch_types=[pltpu.VMEM((ch,), jnp.int32),
                       pltpu.VMEM((ch,), jnp.int32),
                       pltpu.VMEM((ch,), jnp.int32),
                       pltpu.VMEM((ch,), jnp.int32),
                       pltpu.VMEM((ch, d), jnp.float32),
                       pltpu.VMEM((ch, d), jnp.float32),
                       pltpu.VMEM((ch, d), jnp.float32),
                       pltpu.VMEM((ch, d), jnp.float32),
                       pltpu.SemaphoreType.DMA,
                       pltpu.SemaphoreType.DMA,
                       pltpu.SemaphoreType.DMA,
                       pltpu.SemaphoreType.DMA],
    )
    def combine_k(wout_hbm, p0_hbm, p1_hbm, out_hbm,
                  p0a, p0b, p1a, p1b, r0a, r0b, r1a, r1b, g0, g1, w0, w1):
        p0, p1 = [p0a, p0b], [p1a, p1b]
        r0, r1, gs, ws = [r0a, r0b], [r1a, r1b], [g0, g1], [w0, w1]
        wid = lax.axis_index("s") * NC + lax.axis_index("c")
        base = wid * t_per_w
        g0d, g1d, wd = [None] * n, [None] * n, [None] * n
        pltpu.sync_copy(p0_hbm.at[pl.ds(base, ch)], p0[0])
        pltpu.sync_copy(p1_hbm.at[pl.ds(base, ch)], p1[0])
        g0d[0] = pltpu.async_copy(wout_hbm.at[p0[0]], r0[0], gs[0])
        g1d[0] = pltpu.async_copy(wout_hbm.at[p1[0]], r1[0], gs[0])
        for c in range(n):
            cur = c & 1
            nxt = 1 - cur
            if c + 1 < n:
                off_n = base + (c + 1) * ch
                pltpu.sync_copy(p0_hbm.at[pl.ds(off_n, ch)], p0[nxt])
                pltpu.sync_copy(p1_hbm.at[pl.ds(off_n, ch)], p1[nxt])
                if c >= 1:
                    wd[c - 1].wait()
                g0d[c + 1] = pltpu.async_copy(wout_hbm.at[p0[nxt]], r0[nxt],
                                              gs[nxt])
                g1d[c + 1] = pltpu.async_copy(wout_hbm.at[p1[nxt]], r1[nxt],
                                              gs[nxt])
            g0d[c].wait()
            g1d[c].wait()
            for r in range(ch):
                def add_body(ci, _, r=r, cur=cur):
                    sl = pl.ds(ci * LANES, LANES)
                    r0[cur][r, sl] = r0[cur][r, sl] + r1[cur][r, sl]
                    return 0
                lax.fori_loop(0, d // LANES, add_body, 0)
            wd[c] = pltpu.async_copy(r0[cur],
                                     out_hbm.at[pl.ds(base + c * ch, ch)],
                                     ws[cur])
        if n >= 2:
            wd[n - 2].wait()
        wd[n - 1].wait()

    return combine_k(wout, pos0, pos1)


def kernel(hidden_states, gate_w, Wg, Wu, Wd):
    b, s, d = hidden_states.shape
    e, h, _ = Wg.shape
    t = b * s
    p = t * TOPK
    nb = (p + e * (BLK - 1) + BLK - 1) // BLK
    npad = nb * BLK

    x = hidden_states.reshape(t, d)

    logits, topw, topi, x_pk = pl.pallas_call(
        _router_body,
        grid=(t // RBLK,),
        in_specs=[pl.BlockSpec((RBLK, d), lambda i: (i, 0)),
                  pl.BlockSpec((e, d), lambda i: (0, 0))],
        out_specs=[pl.BlockSpec((RBLK, e), lambda i: (i, 0)),
                   pl.BlockSpec((RBLK, TOPK), lambda i: (i, 0)),
                   pl.BlockSpec((RBLK, TOPK), lambda i: (i, 0)),
                   pl.BlockSpec((RBLK, d // 2), lambda i: (i, 0))],
        out_shape=[jax.ShapeDtypeStruct((t, e), jnp.float32),
                   jax.ShapeDtypeStruct((t, TOPK), jnp.float32),
                   jax.ShapeDtypeStruct((t, TOPK), jnp.int32),
                   jax.ShapeDtypeStruct((t, d // 2), jnp.int32)],
    )(x, gate_w)

    pair_e = topi.reshape(-1)
    pair_w = topw.reshape(-1)
    onehot = (pair_e[:, None] ==
              jnp.arange(e, dtype=jnp.int32)[None, :]).astype(jnp.int32)
    csum = jnp.cumsum(onehot, axis=0)
    counts = csum[-1]
    pad_counts = ((counts + BLK - 1) // BLK) * BLK
    ends = jnp.cumsum(pad_counts)
    pad_off = ends - pad_counts
    rank = jnp.take_along_axis(csum, pair_e[:, None], axis=1)[:, 0] - 1
    dest = pad_off[pair_e] + rank
    row_token = jnp.zeros((npad,), jnp.int32).at[dest].set(
        jnp.arange(p, dtype=jnp.int32) // TOPK)
    row_w = jnp.zeros((npad,), jnp.float32).at[dest].set(pair_w)
    pos0 = dest[0::TOPK]
    pos1 = dest[1::TOPK]
    total = ends[-1]
    bstart = jnp.arange(nb, dtype=jnp.int32) * BLK
    block_expert = jnp.minimum(
        jnp.searchsorted(ends, bstart, side="right").astype(jnp.int32), e - 1)
    block_valid = (bstart < total).astype(jnp.int32)

    half_nb = nb // 2
    half_rows = half_nb * BLK
    row_w2 = row_w.reshape(npad, 1)

    xs_a = _sc_gather(x_pk, row_token[:half_rows], half_rows)
    xs_b = _sc_gather(x_pk, row_token[half_rows:], half_rows)

    def mlp_specs(off):
        return [
            pl.BlockSpec((BLK, d // 2), lambda i, be, bv: (i, 0)),
            pl.BlockSpec((1, h, d), lambda i, be, bv: (be[i], 0, 0)),
            pl.BlockSpec((1, h, d), lambda i, be, bv: (be[i], 0, 0)),
            pl.BlockSpec((1, d, h), lambda i, be, bv: (be[i], 0, 0)),
            pl.BlockSpec((BLK, 1), lambda i, be, bv: (i + off, 0)),
        ], pl.BlockSpec((BLK, d), lambda i, be, bv: (i + off, 0))

    in_a, out_a = mlp_specs(0)
    wout_a = pl.pallas_call(
        _mlp_body,
        grid_spec=pltpu.PrefetchScalarGridSpec(
            num_scalar_prefetch=2, grid=(half_nb,),
            in_specs=in_a, out_specs=out_a),
        out_shape=jax.ShapeDtypeStruct((npad, d), jnp.float32),
    )(block_expert[:half_nb], block_valid[:half_nb], xs_a, Wg, Wu, Wd,
      row_w2)

    in_b, out_b = mlp_specs(half_nb)
    in_b = [pl.BlockSpec(memory_space=pl.ANY)] + in_b
    wout = pl.pallas_call(
        _mlp_body_cont,
        grid_spec=pltpu.PrefetchScalarGridSpec(
            num_scalar_prefetch=2, grid=(half_nb,),
            in_specs=in_b, out_specs=out_b),
        out_shape=jax.ShapeDtypeStruct((npad, d), jnp.float32),
        input_output_aliases={2: 0},
    )(block_expert[half_nb:], block_valid[half_nb:], wout_a, xs_b,
      Wg, Wu, Wd, row_w2)

    final = _sc_combine(wout, pos0, pos1)

    return final.reshape(b, s, d), logits


# speedup vs baseline: 1.8366x; 1.3956x over previous
import functools

import jax
import jax.numpy as jnp
from jax import lax
from jax.experimental import pallas as pl
from jax.experimental.pallas import tpu as pltpu
from jax.experimental.pallas import tpu_sc as plsc

TOPK = 2
BLK = 256
RBLK = 512
NC, NS, LANES = 2, 16, 16
NW = NC * NS
MASK_HI = jnp.int32(-65536)

_SC_MESH = dict(core_axis_name="c", subcore_axis_name="s",
                num_cores=NC, num_subcores=NS)


def _router_body(x_ref, gw_ref, logits_ref, topw_ref, topi_ref, xpk_ref):
    x = x_ref[...]
    logits = lax.dot_general(x, gw_ref[...], (((1,), (1,)), ((), ())),
                             preferred_element_type=jnp.float32)
    logits_ref[...] = logits
    d2 = x.shape[1] // 2
    lo = x[:, :d2].astype(jnp.bfloat16).astype(jnp.float32)
    hi = x[:, d2:].astype(jnp.bfloat16).astype(jnp.float32)
    lo_u = lax.bitcast_convert_type(lo, jnp.uint32) >> 16
    hi_u = lax.bitcast_convert_type(hi, jnp.uint32) & jnp.uint32(0xFFFF0000)
    xpk_ref[...] = lax.bitcast_convert_type(hi_u | lo_u, jnp.int32)
    e = logits.shape[1]
    m = jnp.max(logits, axis=1, keepdims=True)
    p = jnp.exp(logits - m)
    probs = p / jnp.sum(p, axis=1, keepdims=True)
    iota = lax.broadcasted_iota(jnp.int32, probs.shape, 1)
    m1 = jnp.max(probs, axis=1, keepdims=True)
    i1 = jnp.min(jnp.where(probs == m1, iota, e), axis=1, keepdims=True)
    probs2 = jnp.where(iota == i1, -jnp.inf, probs)
    m2 = jnp.max(probs2, axis=1, keepdims=True)
    i2 = jnp.min(jnp.where(probs2 == m2, iota, e), axis=1, keepdims=True)
    topw_ref[...] = jnp.concatenate([m1, m2], axis=1)
    topi_ref[...] = jnp.concatenate([i1, i2], axis=1)


def _mlp_body(be_ref, bv_ref, xs_ref, wg_ref, wu_ref, wd_ref, w_ref, out_ref):
    i = pl.program_id(0)

    @pl.when(bv_ref[i] != 0)
    def _():
        xi = lax.bitcast_convert_type(xs_ref[...], jnp.uint32)
        x_lo = lax.bitcast_convert_type(xi << 16, jnp.float32)
        x_hi = lax.bitcast_convert_type(xi & jnp.uint32(0xFFFF0000),
                                        jnp.float32)
        d2 = xi.shape[1]
        dn = (((1,), (1,)), ((), ()))
        wg, wu, wd = wg_ref[0], wu_ref[0], wd_ref[0]
        g = (lax.dot_general(x_lo, wg[:, :d2], dn,
                             preferred_element_type=jnp.float32)
             + lax.dot_general(x_hi, wg[:, d2:], dn,
                               preferred_element_type=jnp.float32))
        u = (lax.dot_general(x_lo, wu[:, :d2], dn,
                             preferred_element_type=jnp.float32)
             + lax.dot_general(x_hi, wu[:, d2:], dn,
                               preferred_element_type=jnp.float32))
        hv = (g * jax.nn.sigmoid(g)) * u
        o = lax.dot_general(hv, wd, dn, preferred_element_type=jnp.float32)
        ow = o * w_ref[...]
        olo = ow[:, :d2].astype(jnp.bfloat16).astype(jnp.float32)
        ohi = ow[:, d2:].astype(jnp.bfloat16).astype(jnp.float32)
        olo_u = lax.bitcast_convert_type(olo, jnp.uint32) >> 16
        ohi_u = (lax.bitcast_convert_type(ohi, jnp.uint32)
                 & jnp.uint32(0xFFFF0000))
        out_ref[...] = lax.bitcast_convert_type(ohi_u | olo_u, jnp.int32)


def _sc_dispatch(x_pk, dest0, dest1, dest, pair_w, npad):
    t, d2 = x_pk.shape
    t_per_w = t // NW
    ct = 32
    nc = t_per_w // ct
    mesh = plsc.VectorSubcoreMesh(**_SC_MESH)

    @functools.partial(
        pl.kernel, mesh=mesh,
        out_type=(jax.ShapeDtypeStruct((npad, d2), jnp.int32),
                  jax.ShapeDtypeStruct((npad,), jnp.float32)),
        scratch_types=[pltpu.VMEM((ct, d2), jnp.int32),
                       pltpu.VMEM((ct, d2), jnp.int32),
                       pltpu.VMEM((ct,), jnp.int32),
                       pltpu.VMEM((ct,), jnp.int32),
                       pltpu.VMEM((ct,), jnp.int32),
                       pltpu.VMEM((ct,), jnp.int32),
                       pltpu.VMEM((2 * ct,), jnp.int32),
                       pltpu.VMEM((2 * ct,), jnp.int32),
                       pltpu.VMEM((2 * ct,), jnp.float32),
                       pltpu.VMEM((2 * ct,), jnp.float32),
                       pltpu.SemaphoreType.DMA,
                       pltpu.SemaphoreType.DMA],
    )
    def dispatch_k(x_hbm, d0_hbm, d1_hbm, dp_hbm, pw_hbm, xs_hbm, rw_hbm,
                   rows0, rows1, d0a, d0b, d1a, d1b, dwa, dwb, wva, wvb,
                   s0, s1):
        rows, d0v, d1v = [rows0, rows1], [d0a, d0b], [d1a, d1b]
        dwv, wvv, sems = [dwa, dwb], [wva, wvb], [s0, s1]
        wid = lax.axis_index("s") * NC + lax.axis_index("c")
        base = wid * t_per_w
        descs = [None] * nc
        for c in range(nc):
            cur = c & 1
            if c >= 2:
                for dsc in descs[c - 2]:
                    dsc.wait()
            toff = base + c * ct
            pltpu.sync_copy(x_hbm.at[pl.ds(toff, ct)], rows[cur])
            pltpu.sync_copy(d0_hbm.at[pl.ds(toff, ct)], d0v[cur])
            pltpu.sync_copy(d1_hbm.at[pl.ds(toff, ct)], d1v[cur])
            pltpu.sync_copy(dp_hbm.at[pl.ds(2 * toff, 2 * ct)], dwv[cur])
            pltpu.sync_copy(pw_hbm.at[pl.ds(2 * toff, 2 * ct)], wvv[cur])
            descs[c] = [
                pltpu.async_copy(rows[cur], xs_hbm.at[d0v[cur]], sems[cur]),
                pltpu.async_copy(rows[cur], xs_hbm.at[d1v[cur]], sems[cur]),
                pltpu.async_copy(wvv[cur], rw_hbm.at[dwv[cur]], sems[cur]),
            ]
        for c in (nc - 2, nc - 1):
            if c >= 0:
                for dsc in descs[c]:
                    dsc.wait()

    return dispatch_k(x_pk, dest0, dest1, dest, pair_w)


def _sc_combine(wout_pk, pos0, pos1, d):
    t = pos0.shape[0]
    d2 = wout_pk.shape[1]
    t_per_w = t // NW
    ch = 8
    n = t_per_w // ch
    mesh = plsc.VectorSubcoreMesh(**_SC_MESH)

    @functools.partial(
        pl.kernel, mesh=mesh,
        out_type=jax.ShapeDtypeStruct((t, d), jnp.float32),
        scratch_types=[pltpu.VMEM((ch,), jnp.int32),
                       pltpu.VMEM((ch,), jnp.int32),
                       pltpu.VMEM((ch,), jnp.int32),
                       pltpu.VMEM((ch,), jnp.int32),
                       pltpu.VMEM((ch, d2), jnp.int32),
                       pltpu.VMEM((ch, d2), jnp.int32),
                       pltpu.VMEM((ch, d2), jnp.int32),
                       pltpu.VMEM((ch, d2), jnp.int32),
                       pltpu.VMEM((ch, d), jnp.float32),
                       pltpu.VMEM((ch, d), jnp.float32),
                       pltpu.SemaphoreType.DMA,
                       pltpu.SemaphoreType.DMA,
                       pltpu.SemaphoreType.DMA,
                       pltpu.SemaphoreType.DMA],
    )
    def combine_k(wout_hbm, p0_hbm, p1_hbm, out_hbm,
                  p0a, p0b, p1a, p1b, r0a, r0b, r1a, r1b, oba, obb,
                  g0, g1, w0, w1):
        p0, p1 = [p0a, p0b], [p1a, p1b]
        r0, r1, ob = [r0a, r0b], [r1a, r1b], [oba, obb]
        gs, ws = [g0, g1], [w0, w1]
        wid = lax.axis_index("s") * NC + lax.axis_index("c")
        base = wid * t_per_w
        g0d, g1d, wd = [None] * n, [None] * n, [None] * n
        pltpu.sync_copy(p0_hbm.at[pl.ds(base, ch)], p0[0])
        pltpu.sync_copy(p1_hbm.at[pl.ds(base, ch)], p1[0])
        g0d[0] = pltpu.async_copy(wout_hbm.at[p0[0]], r0[0], gs[0])
        g1d[0] = pltpu.async_copy(wout_hbm.at[p1[0]], r1[0], gs[0])
        for c in range(n):
            cur = c & 1
            nxt = 1 - cur
            if c + 1 < n:
                off_n = base + (c + 1) * ch
                pltpu.sync_copy(p0_hbm.at[pl.ds(off_n, ch)], p0[nxt])
                pltpu.sync_copy(p1_hbm.at[pl.ds(off_n, ch)], p1[nxt])
                if c >= 1:
                    wd[c - 1].wait()
                g0d[c + 1] = pltpu.async_copy(wout_hbm.at[p0[nxt]], r0[nxt],
                                              gs[nxt])
                g1d[c + 1] = pltpu.async_copy(wout_hbm.at[p1[nxt]], r1[nxt],
                                              gs[nxt])
            g0d[c].wait()
            g1d[c].wait()
            for r in range(ch):
                def add_body(ci, _, r=r, cur=cur):
                    sl = pl.ds(ci * LANES, LANES)
                    v0 = r0[cur][r, sl]
                    v1 = r1[cur][r, sl]
                    bc = lambda z: lax.bitcast_convert_type(z, jnp.float32)
                    lo = bc(v0 << 16) + bc(v1 << 16)
                    hi = bc(v0 & MASK_HI) + bc(v1 & MASK_HI)
                    ob[cur][r, sl] = lo
                    ob[cur][r, pl.ds(d2 + ci * LANES, LANES)] = hi
                    return 0
                lax.fori_loop(0, d2 // LANES, add_body, 0)
            wd[c] = pltpu.async_copy(ob[cur],
                                     out_hbm.at[pl.ds(base + c * ch, ch)],
                                     ws[cur])
        if n >= 2:
            wd[n - 2].wait()
        wd[n - 1].wait()

    return combine_k(wout_pk, pos0, pos1)


def kernel(hidden_states, gate_w, Wg, Wu, Wd):
    b, s, d = hidden_states.shape
    e, h, _ = Wg.shape
    t = b * s
    p = t * TOPK
    nb = (p + e * (BLK - 1) + BLK - 1) // BLK
    npad = nb * BLK

    x = hidden_states.reshape(t, d)

    logits, topw, topi, x_pk = pl.pallas_call(
        _router_body,
        grid=(t // RBLK,),
        in_specs=[pl.BlockSpec((RBLK, d), lambda i: (i, 0)),
                  pl.BlockSpec((e, d), lambda i: (0, 0))],
        out_specs=[pl.BlockSpec((RBLK, e), lambda i: (i, 0)),
                   pl.BlockSpec((RBLK, TOPK), lambda i: (i, 0)),
                   pl.BlockSpec((RBLK, TOPK), lambda i: (i, 0)),
                   pl.BlockSpec((RBLK, d // 2), lambda i: (i, 0))],
        out_shape=[jax.ShapeDtypeStruct((t, e), jnp.float32),
                   jax.ShapeDtypeStruct((t, TOPK), jnp.float32),
                   jax.ShapeDtypeStruct((t, TOPK), jnp.int32),
                   jax.ShapeDtypeStruct((t, d // 2), jnp.int32)],
    )(x, gate_w)

    pair_e = topi.reshape(-1)
    pair_w = topw.reshape(-1)
    onehot = (pair_e[:, None] ==
              jnp.arange(e, dtype=jnp.int32)[None, :]).astype(jnp.int32)
    csum = onehot
    k = 1
    while k < p:
        csum = csum + jnp.concatenate(
            [jnp.zeros((k, e), jnp.int32), csum[:-k]], axis=0)
        k *= 2
    counts = csum[-1]
    pad_counts = ((counts + BLK - 1) // BLK) * BLK
    ends = jnp.cumsum(pad_counts)
    pad_off = ends - pad_counts
    rank = jnp.sum(onehot * csum, axis=1) - 1
    dest = (jnp.sum(onehot * pad_off[None, :], axis=1) + rank
            ).astype(jnp.int32)
    pos0 = dest[0::TOPK]
    pos1 = dest[1::TOPK]
    total = ends[-1]
    bstart = jnp.arange(nb, dtype=jnp.int32) * BLK
    block_expert = jnp.minimum(
        jnp.sum((bstart[:, None] >= ends[None, :]).astype(jnp.int32),
                axis=1), e - 1).astype(jnp.int32)
    block_valid = (bstart < total).astype(jnp.int32)

    xs, row_w = _sc_dispatch(x_pk, pos0, pos1, dest, pair_w, npad)

    grid_spec = pltpu.PrefetchScalarGridSpec(
        num_scalar_prefetch=2,
        grid=(nb,),
        in_specs=[
            pl.BlockSpec((BLK, d // 2), lambda i, be, bv: (i, 0)),
            pl.BlockSpec((1, h, d), lambda i, be, bv: (be[i], 0, 0)),
            pl.BlockSpec((1, h, d), lambda i, be, bv: (be[i], 0, 0)),
            pl.BlockSpec((1, d, h), lambda i, be, bv: (be[i], 0, 0)),
            pl.BlockSpec((BLK, 1), lambda i, be, bv: (i, 0)),
        ],
        out_specs=pl.BlockSpec((BLK, d // 2), lambda i, be, bv: (i, 0)),
    )
    wout_pk = pl.pallas_call(
        _mlp_body,
        grid_spec=grid_spec,
        out_shape=jax.ShapeDtypeStruct((npad, d // 2), jnp.int32),
    )(block_expert, block_valid, xs, Wg, Wu, Wd, row_w.reshape(npad, 1))

    final = _sc_combine(wout_pk, pos0, pos1, d)

    return final.reshape(b, s, d), logits


# speedup vs baseline: 1.8716x; 1.0191x over previous
import functools

import jax
import jax.numpy as jnp
from jax import lax
from jax.experimental import pallas as pl
from jax.experimental.pallas import tpu as pltpu
from jax.experimental.pallas import tpu_sc as plsc

TOPK = 2
BLK = 256
RBLK = 512
NC, NS, LANES = 2, 16, 16
NW = NC * NS
MASK_HI = jnp.int32(-65536)

_SC_MESH = dict(core_axis_name="c", subcore_axis_name="s",
                num_cores=NC, num_subcores=NS)


def _router_body(x_ref, gw_ref, logits_ref, topw_ref, topi_ref, xpk_ref):
    x = x_ref[...]
    logits = lax.dot_general(x, gw_ref[...], (((1,), (1,)), ((), ())),
                             preferred_element_type=jnp.float32)
    logits_ref[...] = logits
    d2 = x.shape[1] // 2
    lo = x[:, :d2].astype(jnp.bfloat16).astype(jnp.float32)
    hi = x[:, d2:].astype(jnp.bfloat16).astype(jnp.float32)
    lo_u = lax.bitcast_convert_type(lo, jnp.uint32) >> 16
    hi_u = lax.bitcast_convert_type(hi, jnp.uint32) & jnp.uint32(0xFFFF0000)
    xpk_ref[...] = lax.bitcast_convert_type(hi_u | lo_u, jnp.int32)
    e = logits.shape[1]
    m = jnp.max(logits, axis=1, keepdims=True)
    p = jnp.exp(logits - m)
    probs = p / jnp.sum(p, axis=1, keepdims=True)
    iota = lax.broadcasted_iota(jnp.int32, probs.shape, 1)
    m1 = jnp.max(probs, axis=1, keepdims=True)
    i1 = jnp.min(jnp.where(probs == m1, iota, e), axis=1, keepdims=True)
    probs2 = jnp.where(iota == i1, -jnp.inf, probs)
    m2 = jnp.max(probs2, axis=1, keepdims=True)
    i2 = jnp.min(jnp.where(probs2 == m2, iota, e), axis=1, keepdims=True)
    topw_ref[...] = jnp.concatenate([m1, m2], axis=1)
    topi_ref[...] = jnp.concatenate([i1, i2], axis=1)


def _mlp_body(be_ref, bv_ref, xs_ref, wg_ref, wu_ref, wd_ref, w_ref, out_ref):
    i = pl.program_id(0)

    @pl.when(bv_ref[i] != 0)
    def _():
        xi = lax.bitcast_convert_type(xs_ref[...], jnp.uint32)
        x_lo = lax.bitcast_convert_type(xi << 16, jnp.float32)
        x_hi = lax.bitcast_convert_type(xi & jnp.uint32(0xFFFF0000),
                                        jnp.float32)
        d2 = xi.shape[1]
        dn = (((1,), (1,)), ((), ()))
        dot = functools.partial(lax.dot_general,
                                precision=lax.Precision.DEFAULT,
                                preferred_element_type=jnp.float32)
        wg, wu, wd = wg_ref[0], wu_ref[0], wd_ref[0]
        g = dot(x_lo, wg[:, :d2], dn) + dot(x_hi, wg[:, d2:], dn)
        u = dot(x_lo, wu[:, :d2], dn) + dot(x_hi, wu[:, d2:], dn)
        hv = (g * jax.nn.sigmoid(g)) * u
        o = dot(hv, wd, dn)
        ow = o * w_ref[...]
        olo = ow[:, :d2].astype(jnp.bfloat16).astype(jnp.float32)
        ohi = ow[:, d2:].astype(jnp.bfloat16).astype(jnp.float32)
        olo_u = lax.bitcast_convert_type(olo, jnp.uint32) >> 16
        ohi_u = (lax.bitcast_convert_type(ohi, jnp.uint32)
                 & jnp.uint32(0xFFFF0000))
        out_ref[...] = lax.bitcast_convert_type(ohi_u | olo_u, jnp.int32)


def _sc_dispatch(x_pk, dest0, dest1, dest, pair_w, npad):
    t, d2 = x_pk.shape
    t_per_w = t // NW
    ct = 32
    nc = t_per_w // ct
    mesh = plsc.VectorSubcoreMesh(**_SC_MESH)

    @functools.partial(
        pl.kernel, mesh=mesh,
        out_type=(jax.ShapeDtypeStruct((npad, d2), jnp.int32),
                  jax.ShapeDtypeStruct((npad,), jnp.float32)),
        scratch_types=[pltpu.VMEM((ct, d2), jnp.int32),
                       pltpu.VMEM((ct, d2), jnp.int32),
                       pltpu.VMEM((ct,), jnp.int32),
                       pltpu.VMEM((ct,), jnp.int32),
                       pltpu.VMEM((ct,), jnp.int32),
                       pltpu.VMEM((ct,), jnp.int32),
                       pltpu.VMEM((2 * ct,), jnp.int32),
                       pltpu.VMEM((2 * ct,), jnp.int32),
                       pltpu.VMEM((2 * ct,), jnp.float32),
                       pltpu.VMEM((2 * ct,), jnp.float32),
                       pltpu.SemaphoreType.DMA,
                       pltpu.SemaphoreType.DMA],
    )
    def dispatch_k(x_hbm, d0_hbm, d1_hbm, dp_hbm, pw_hbm, xs_hbm, rw_hbm,
                   rows0, rows1, d0a, d0b, d1a, d1b, dwa, dwb, wva, wvb,
                   s0, s1):
        rows, d0v, d1v = [rows0, rows1], [d0a, d0b], [d1a, d1b]
        dwv, wvv, sems = [dwa, dwb], [wva, wvb], [s0, s1]
        wid = lax.axis_index("s") * NC + lax.axis_index("c")
        base = wid * t_per_w
        descs = [None] * nc
        for c in range(nc):
            cur = c & 1
            if c >= 2:
                for dsc in descs[c - 2]:
                    dsc.wait()
            toff = base + c * ct
            pltpu.sync_copy(x_hbm.at[pl.ds(toff, ct)], rows[cur])
            pltpu.sync_copy(d0_hbm.at[pl.ds(toff, ct)], d0v[cur])
            pltpu.sync_copy(d1_hbm.at[pl.ds(toff, ct)], d1v[cur])
            pltpu.sync_copy(dp_hbm.at[pl.ds(2 * toff, 2 * ct)], dwv[cur])
            pltpu.sync_copy(pw_hbm.at[pl.ds(2 * toff, 2 * ct)], wvv[cur])
            descs[c] = [
                pltpu.async_copy(rows[cur], xs_hbm.at[d0v[cur]], sems[cur]),
                pltpu.async_copy(rows[cur], xs_hbm.at[d1v[cur]], sems[cur]),
                pltpu.async_copy(wvv[cur], rw_hbm.at[dwv[cur]], sems[cur]),
            ]
        for c in (nc - 2, nc - 1):
            if c >= 0:
                for dsc in descs[c]:
                    dsc.wait()

    return dispatch_k(x_pk, dest0, dest1, dest, pair_w)


def _sc_combine(wout_pk, pos0, pos1, d):
    t = pos0.shape[0]
    d2 = wout_pk.shape[1]
    t_per_w = t // NW
    ch = 8
    n = t_per_w // ch
    mesh = plsc.VectorSubcoreMesh(**_SC_MESH)

    @functools.partial(
        pl.kernel, mesh=mesh,
        out_type=jax.ShapeDtypeStruct((t, d), jnp.float32),
        scratch_types=[pltpu.VMEM((ch,), jnp.int32),
                       pltpu.VMEM((ch,), jnp.int32),
                       pltpu.VMEM((ch,), jnp.int32),
                       pltpu.VMEM((ch,), jnp.int32),
                       pltpu.VMEM((ch, d2), jnp.int32),
                       pltpu.VMEM((ch, d2), jnp.int32),
                       pltpu.VMEM((ch, d2), jnp.int32),
                       pltpu.VMEM((ch, d2), jnp.int32),
                       pltpu.VMEM((ch, d), jnp.float32),
                       pltpu.VMEM((ch, d), jnp.float32),
                       pltpu.SemaphoreType.DMA,
                       pltpu.SemaphoreType.DMA,
                       pltpu.SemaphoreType.DMA,
                       pltpu.SemaphoreType.DMA],
    )
    def combine_k(wout_hbm, p0_hbm, p1_hbm, out_hbm,
                  p0a, p0b, p1a, p1b, r0a, r0b, r1a, r1b, oba, obb,
                  g0, g1, w0, w1):
        p0, p1 = [p0a, p0b], [p1a, p1b]
        r0, r1, ob = [r0a, r0b], [r1a, r1b], [oba, obb]
        gs, ws = [g0, g1], [w0, w1]
        wid = lax.axis_index("s") * NC + lax.axis_index("c")
        base = wid * t_per_w
        g0d, g1d, wd = [None] * n, [None] * n, [None] * n
        pltpu.sync_copy(p0_hbm.at[pl.ds(base, ch)], p0[0])
        pltpu.sync_copy(p1_hbm.at[pl.ds(base, ch)], p1[0])
        g0d[0] = pltpu.async_copy(wout_hbm.at[p0[0]], r0[0], gs[0])
        g1d[0] = pltpu.async_copy(wout_hbm.at[p1[0]], r1[0], gs[0])
        for c in range(n):
            cur = c & 1
            nxt = 1 - cur
            if c + 1 < n:
                off_n = base + (c + 1) * ch
                pltpu.sync_copy(p0_hbm.at[pl.ds(off_n, ch)], p0[nxt])
                pltpu.sync_copy(p1_hbm.at[pl.ds(off_n, ch)], p1[nxt])
                if c >= 1:
                    wd[c - 1].wait()
                g0d[c + 1] = pltpu.async_copy(wout_hbm.at[p0[nxt]], r0[nxt],
                                              gs[nxt])
                g1d[c + 1] = pltpu.async_copy(wout_hbm.at[p1[nxt]], r1[nxt],
                                              gs[nxt])
            g0d[c].wait()
            g1d[c].wait()
            for r in range(ch):
                def add_body(ci, _, r=r, cur=cur):
                    sl = pl.ds(ci * LANES, LANES)
                    v0 = r0[cur][r, sl]
                    v1 = r1[cur][r, sl]
                    bc = lambda z: lax.bitcast_convert_type(z, jnp.float32)
                    lo = bc(v0 << 16) + bc(v1 << 16)
                    hi = bc(v0 & MASK_HI) + bc(v1 & MASK_HI)
                    ob[cur][r, sl] = lo
                    ob[cur][r, pl.ds(d2 + ci * LANES, LANES)] = hi
                    return 0
                lax.fori_loop(0, d2 // LANES, add_body, 0)
            wd[c] = pltpu.async_copy(ob[cur],
                                     out_hbm.at[pl.ds(base + c * ch, ch)],
                                     ws[cur])
        if n >= 2:
            wd[n - 2].wait()
        wd[n - 1].wait()

    return combine_k(wout_pk, pos0, pos1)


def kernel(hidden_states, gate_w, Wg, Wu, Wd):
    b, s, d = hidden_states.shape
    e, h, _ = Wg.shape
    t = b * s
    p = t * TOPK
    nb = (p + e * (BLK - 1) + BLK - 1) // BLK
    npad = nb * BLK

    x = hidden_states.reshape(t, d)

    logits, topw, topi, x_pk = pl.pallas_call(
        _router_body,
        grid=(t // RBLK,),
        in_specs=[pl.BlockSpec((RBLK, d), lambda i: (i, 0)),
                  pl.BlockSpec((e, d), lambda i: (0, 0))],
        out_specs=[pl.BlockSpec((RBLK, e), lambda i: (i, 0)),
                   pl.BlockSpec((RBLK, TOPK), lambda i: (i, 0)),
                   pl.BlockSpec((RBLK, TOPK), lambda i: (i, 0)),
                   pl.BlockSpec((RBLK, d // 2), lambda i: (i, 0))],
        out_shape=[jax.ShapeDtypeStruct((t, e), jnp.float32),
                   jax.ShapeDtypeStruct((t, TOPK), jnp.float32),
                   jax.ShapeDtypeStruct((t, TOPK), jnp.int32),
                   jax.ShapeDtypeStruct((t, d // 2), jnp.int32)],
    )(x, gate_w)

    pair_e = topi.reshape(-1)
    pair_w = topw.reshape(-1)
    onehot = (pair_e[:, None] ==
              jnp.arange(e, dtype=jnp.int32)[None, :]).astype(jnp.int32)
    csum = onehot
    k = 1
    while k < p:
        csum = csum + jnp.concatenate(
            [jnp.zeros((k, e), jnp.int32), csum[:-k]], axis=0)
        k *= 2
    counts = csum[-1]
    pad_counts = ((counts + BLK - 1) // BLK) * BLK
    ends = jnp.cumsum(pad_counts)
    pad_off = ends - pad_counts
    rank = jnp.sum(onehot * csum, axis=1) - 1
    dest = (jnp.sum(onehot * pad_off[None, :], axis=1) + rank
            ).astype(jnp.int32)
    pos0 = dest[0::TOPK]
    pos1 = dest[1::TOPK]
    total = ends[-1]
    bstart = jnp.arange(nb, dtype=jnp.int32) * BLK
    block_expert = jnp.minimum(
        jnp.sum((bstart[:, None] >= ends[None, :]).astype(jnp.int32),
                axis=1), e - 1).astype(jnp.int32)
    block_valid = (bstart < total).astype(jnp.int32)

    xs, row_w = _sc_dispatch(x_pk, pos0, pos1, dest, pair_w, npad)

    grid_spec = pltpu.PrefetchScalarGridSpec(
        num_scalar_prefetch=2,
        grid=(nb,),
        in_specs=[
            pl.BlockSpec((BLK, d // 2), lambda i, be, bv: (i, 0)),
            pl.BlockSpec((1, h, d), lambda i, be, bv: (be[i], 0, 0)),
            pl.BlockSpec((1, h, d), lambda i, be, bv: (be[i], 0, 0)),
            pl.BlockSpec((1, d, h), lambda i, be, bv: (be[i], 0, 0)),
            pl.BlockSpec((BLK, 1), lambda i, be, bv: (i, 0)),
        ],
        out_specs=pl.BlockSpec((BLK, d // 2), lambda i, be, bv: (i, 0)),
    )
    wout_pk = pl.pallas_call(
        _mlp_body,
        grid_spec=grid_spec,
        out_shape=jax.ShapeDtypeStruct((npad, d // 2), jnp.int32),
    )(block_expert, block_valid, xs, Wg, Wu, Wd, row_w.reshape(npad, 1))

    final = _sc_combine(wout_pk, pos0, pos1, d)

    return final.reshape(b, s, d), logits


# speedup vs baseline: 1.8841x; 1.0067x over previous
import functools

import jax
import jax.numpy as jnp
from jax import lax
from jax.experimental import pallas as pl
from jax.experimental.pallas import tpu as pltpu
from jax.experimental.pallas import tpu_sc as plsc

TOPK = 2
BLK = 256
RBLK = 512
NC, NS, LANES = 2, 16, 16
NW = NC * NS
MASK_HI = jnp.int32(-65536)

_SC_MESH = dict(core_axis_name="c", subcore_axis_name="s",
                num_cores=NC, num_subcores=NS)


def _router_body(x_ref, gw_ref, logits_ref, topw_ref, topi_ref, xpk_ref,
                 rank_ref, cnt_out_ref, cnt_ref):
    i = pl.program_id(0)
    x = x_ref[...]
    logits = lax.dot_general(x, gw_ref[...], (((1,), (1,)), ((), ())),
                             preferred_element_type=jnp.float32)
    logits_ref[...] = logits
    d2 = x.shape[1] // 2
    lo = x[:, :d2].astype(jnp.bfloat16).astype(jnp.float32)
    hi = x[:, d2:].astype(jnp.bfloat16).astype(jnp.float32)
    lo_u = lax.bitcast_convert_type(lo, jnp.uint32) >> 16
    hi_u = lax.bitcast_convert_type(hi, jnp.uint32) & jnp.uint32(0xFFFF0000)
    xpk_ref[...] = lax.bitcast_convert_type(hi_u | lo_u, jnp.int32)
    e = logits.shape[1]
    m = jnp.max(logits, axis=1, keepdims=True)
    p = jnp.exp(logits - m)
    probs = p / jnp.sum(p, axis=1, keepdims=True)
    iota = lax.broadcasted_iota(jnp.int32, probs.shape, 1)
    m1 = jnp.max(probs, axis=1, keepdims=True)
    i1 = jnp.min(jnp.where(probs == m1, iota, e), axis=1, keepdims=True)
    probs2 = jnp.where(iota == i1, -jnp.inf, probs)
    m2 = jnp.max(probs2, axis=1, keepdims=True)
    i2 = jnp.min(jnp.where(probs2 == m2, iota, e), axis=1, keepdims=True)
    topw_ref[...] = jnp.concatenate([m1, m2], axis=1)
    topi_ref[...] = jnp.concatenate([i1, i2], axis=1)

    @pl.when(i == 0)
    def _():
        cnt_ref[...] = jnp.zeros_like(cnt_ref)
    oh0 = (i1 == iota).astype(jnp.int32)
    oh1 = (i2 == iota).astype(jnp.int32)
    c01 = oh0 + oh1
    pre = c01
    k = 1
    while k < c01.shape[0]:
        pre = pre + jnp.concatenate(
            [jnp.zeros((k, e), jnp.int32), pre[:-k]], axis=0)
        k *= 2
    pre = pre - c01 + cnt_ref[...]
    rank0 = jnp.sum(oh0 * pre, axis=1, keepdims=True)
    rank1 = jnp.sum(oh1 * (pre + oh0), axis=1, keepdims=True)
    rank_ref[...] = jnp.concatenate([rank0, rank1], axis=1)
    cnt_ref[...] = cnt_ref[...] + jnp.sum(c01, axis=0, keepdims=True)
    cnt_out_ref[...] = cnt_ref[...]


def _mlp_body(be_ref, bv_ref, xs_ref, wg_ref, wu_ref, wd_ref, w_ref, out_ref):
    i = pl.program_id(0)

    @pl.when(bv_ref[i] != 0)
    def _():
        xi = lax.bitcast_convert_type(xs_ref[...], jnp.uint32)
        x_lo = lax.bitcast_convert_type(xi << 16, jnp.float32)
        x_hi = lax.bitcast_convert_type(xi & jnp.uint32(0xFFFF0000),
                                        jnp.float32)
        d2 = xi.shape[1]
        dn = (((1,), (1,)), ((), ()))
        dot = functools.partial(lax.dot_general,
                                precision=lax.Precision.DEFAULT,
                                preferred_element_type=jnp.float32)
        wg, wu, wd = wg_ref[0], wu_ref[0], wd_ref[0]
        g = dot(x_lo, wg[:, :d2], dn) + dot(x_hi, wg[:, d2:], dn)
        u = dot(x_lo, wu[:, :d2], dn) + dot(x_hi, wu[:, d2:], dn)
        hv = (g * jax.nn.sigmoid(g)) * u
        o = dot(hv, wd, dn)
        ow = o * w_ref[...]
        olo = ow[:, :d2].astype(jnp.bfloat16).astype(jnp.float32)
        ohi = ow[:, d2:].astype(jnp.bfloat16).astype(jnp.float32)
        olo_u = lax.bitcast_convert_type(olo, jnp.uint32) >> 16
        ohi_u = (lax.bitcast_convert_type(ohi, jnp.uint32)
                 & jnp.uint32(0xFFFF0000))
        out_ref[...] = lax.bitcast_convert_type(ohi_u | olo_u, jnp.int32)


def _sc_dispatch(x_pk, dest0, dest1, dest, pair_w, npad):
    t, d2 = x_pk.shape
    t_per_w = t // NW
    ct = 32
    nc = t_per_w // ct
    mesh = plsc.VectorSubcoreMesh(**_SC_MESH)

    @functools.partial(
        pl.kernel, mesh=mesh,
        out_type=(jax.ShapeDtypeStruct((npad, d2), jnp.int32),
                  jax.ShapeDtypeStruct((npad,), jnp.float32)),
        scratch_types=[pltpu.VMEM((ct, d2), jnp.int32),
                       pltpu.VMEM((ct, d2), jnp.int32),
                       pltpu.VMEM((ct,), jnp.int32),
                       pltpu.VMEM((ct,), jnp.int32),
                       pltpu.VMEM((ct,), jnp.int32),
                       pltpu.VMEM((ct,), jnp.int32),
                       pltpu.VMEM((2 * ct,), jnp.int32),
                       pltpu.VMEM((2 * ct,), jnp.int32),
                       pltpu.VMEM((2 * ct,), jnp.float32),
                       pltpu.VMEM((2 * ct,), jnp.float32),
                       pltpu.SemaphoreType.DMA,
                       pltpu.SemaphoreType.DMA],
    )
    def dispatch_k(x_hbm, d0_hbm, d1_hbm, dp_hbm, pw_hbm, xs_hbm, rw_hbm,
                   rows0, rows1, d0a, d0b, d1a, d1b, dwa, dwb, wva, wvb,
                   s0, s1):
        rows, d0v, d1v = [rows0, rows1], [d0a, d0b], [d1a, d1b]
        dwv, wvv, sems = [dwa, dwb], [wva, wvb], [s0, s1]
        wid = lax.axis_index("s") * NC + lax.axis_index("c")
        base = wid * t_per_w
        descs = [None] * nc
        for c in range(nc):
            cur = c & 1
            if c >= 2:
                for dsc in descs[c - 2]:
                    dsc.wait()
            toff = base + c * ct
            pltpu.sync_copy(x_hbm.at[pl.ds(toff, ct)], rows[cur])
            pltpu.sync_copy(d0_hbm.at[pl.ds(toff, ct)], d0v[cur])
            pltpu.sync_copy(d1_hbm.at[pl.ds(toff, ct)], d1v[cur])
            pltpu.sync_copy(dp_hbm.at[pl.ds(2 * toff, 2 * ct)], dwv[cur])
            pltpu.sync_copy(pw_hbm.at[pl.ds(2 * toff, 2 * ct)], wvv[cur])
            descs[c] = [
                pltpu.async_copy(rows[cur], xs_hbm.at[d0v[cur]], sems[cur]),
                pltpu.async_copy(rows[cur], xs_hbm.at[d1v[cur]], sems[cur]),
                pltpu.async_copy(wvv[cur], rw_hbm.at[dwv[cur]], sems[cur]),
            ]
        for c in (nc - 2, nc - 1):
            if c >= 0:
                for dsc in descs[c]:
                    dsc.wait()

    return dispatch_k(x_pk, dest0, dest1, dest, pair_w)


def _sc_combine(wout_pk, pos0, pos1, d):
    t = pos0.shape[0]
    d2 = wout_pk.shape[1]
    t_per_w = t // NW
    ch = 8
    n = t_per_w // ch
    mesh = plsc.VectorSubcoreMesh(**_SC_MESH)

    @functools.partial(
        pl.kernel, mesh=mesh,
        out_type=jax.ShapeDtypeStruct((t, d), jnp.float32),
        scratch_types=[pltpu.VMEM((ch,), jnp.int32),
                       pltpu.VMEM((ch,), jnp.int32),
                       pltpu.VMEM((ch,), jnp.int32),
                       pltpu.VMEM((ch,), jnp.int32),
                       pltpu.VMEM((ch, d2), jnp.int32),
                       pltpu.VMEM((ch, d2), jnp.int32),
                       pltpu.VMEM((ch, d2), jnp.int32),
                       pltpu.VMEM((ch, d2), jnp.int32),
                       pltpu.VMEM((ch, d), jnp.float32),
                       pltpu.VMEM((ch, d), jnp.float32),
                       pltpu.SemaphoreType.DMA,
                       pltpu.SemaphoreType.DMA,
                       pltpu.SemaphoreType.DMA,
                       pltpu.SemaphoreType.DMA],
    )
    def combine_k(wout_hbm, p0_hbm, p1_hbm, out_hbm,
                  p0a, p0b, p1a, p1b, r0a, r0b, r1a, r1b, oba, obb,
                  g0, g1, w0, w1):
        p0, p1 = [p0a, p0b], [p1a, p1b]
        r0, r1, ob = [r0a, r0b], [r1a, r1b], [oba, obb]
        gs, ws = [g0, g1], [w0, w1]
        wid = lax.axis_index("s") * NC + lax.axis_index("c")
        base = wid * t_per_w
        g0d, g1d, wd = [None] * n, [None] * n, [None] * n
        pltpu.sync_copy(p0_hbm.at[pl.ds(base, ch)], p0[0])
        pltpu.sync_copy(p1_hbm.at[pl.ds(base, ch)], p1[0])
        g0d[0] = pltpu.async_copy(wout_hbm.at[p0[0]], r0[0], gs[0])
        g1d[0] = pltpu.async_copy(wout_hbm.at[p1[0]], r1[0], gs[0])
        for c in range(n):
            cur = c & 1
            nxt = 1 - cur
            if c + 1 < n:
                off_n = base + (c + 1) * ch
                pltpu.sync_copy(p0_hbm.at[pl.ds(off_n, ch)], p0[nxt])
                pltpu.sync_copy(p1_hbm.at[pl.ds(off_n, ch)], p1[nxt])
                if c >= 1:
                    wd[c - 1].wait()
                g0d[c + 1] = pltpu.async_copy(wout_hbm.at[p0[nxt]], r0[nxt],
                                              gs[nxt])
                g1d[c + 1] = pltpu.async_copy(wout_hbm.at[p1[nxt]], r1[nxt],
                                              gs[nxt])
            g0d[c].wait()
            g1d[c].wait()
            for r in range(ch):
                def add_body(ci, _, r=r, cur=cur):
                    sl = pl.ds(ci * LANES, LANES)
                    v0 = r0[cur][r, sl]
                    v1 = r1[cur][r, sl]
                    bc = lambda z: lax.bitcast_convert_type(z, jnp.float32)
                    lo = bc(v0 << 16) + bc(v1 << 16)
                    hi = bc(v0 & MASK_HI) + bc(v1 & MASK_HI)
                    ob[cur][r, sl] = lo
                    ob[cur][r, pl.ds(d2 + ci * LANES, LANES)] = hi
                    return 0
                lax.fori_loop(0, d2 // LANES, add_body, 0)
            wd[c] = pltpu.async_copy(ob[cur],
                                     out_hbm.at[pl.ds(base + c * ch, ch)],
                                     ws[cur])
        if n >= 2:
            wd[n - 2].wait()
        wd[n - 1].wait()

    return combine_k(wout_pk, pos0, pos1)


def kernel(hidden_states, gate_w, Wg, Wu, Wd):
    b, s, d = hidden_states.shape
    e, h, _ = Wg.shape
    t = b * s
    p = t * TOPK
    nb = (p + e * (BLK - 1) + BLK - 1) // BLK
    npad = nb * BLK

    x = hidden_states.reshape(t, d)

    logits, topw, topi, x_pk, rank, counts = pl.pallas_call(
        _router_body,
        grid=(t // RBLK,),
        in_specs=[pl.BlockSpec((RBLK, d), lambda i: (i, 0)),
                  pl.BlockSpec((e, d), lambda i: (0, 0))],
        out_specs=[pl.BlockSpec((RBLK, e), lambda i: (i, 0)),
                   pl.BlockSpec((RBLK, TOPK), lambda i: (i, 0)),
                   pl.BlockSpec((RBLK, TOPK), lambda i: (i, 0)),
                   pl.BlockSpec((RBLK, d // 2), lambda i: (i, 0)),
                   pl.BlockSpec((RBLK, TOPK), lambda i: (i, 0)),
                   pl.BlockSpec((1, e), lambda i: (0, 0))],
        out_shape=[jax.ShapeDtypeStruct((t, e), jnp.float32),
                   jax.ShapeDtypeStruct((t, TOPK), jnp.float32),
                   jax.ShapeDtypeStruct((t, TOPK), jnp.int32),
                   jax.ShapeDtypeStruct((t, d // 2), jnp.int32),
                   jax.ShapeDtypeStruct((t, TOPK), jnp.int32),
                   jax.ShapeDtypeStruct((1, e), jnp.int32)],
        scratch_shapes=[pltpu.VMEM((1, e), jnp.int32)],
    )(x, gate_w)

    pair_e = topi.reshape(-1)
    pair_w = topw.reshape(-1)
    counts = counts.reshape(e)
    pad_counts = ((counts + BLK - 1) // BLK) * BLK
    ends = jnp.cumsum(pad_counts)
    pad_off = ends - pad_counts
    dest = (pad_off[pair_e] + rank.reshape(-1)).astype(jnp.int32)
    pos0 = dest[0::TOPK]
    pos1 = dest[1::TOPK]
    total = ends[-1]
    bstart = jnp.arange(nb, dtype=jnp.int32) * BLK
    block_expert = jnp.minimum(
        jnp.sum((bstart[:, None] >= ends[None, :]).astype(jnp.int32),
                axis=1), e - 1).astype(jnp.int32)
    block_valid = (bstart < total).astype(jnp.int32)

    xs, row_w = _sc_dispatch(x_pk, pos0, pos1, dest, pair_w, npad)

    grid_spec = pltpu.PrefetchScalarGridSpec(
        num_scalar_prefetch=2,
        grid=(nb,),
        in_specs=[
            pl.BlockSpec((BLK, d // 2), lambda i, be, bv: (i, 0)),
            pl.BlockSpec((1, h, d), lambda i, be, bv: (be[i], 0, 0)),
            pl.BlockSpec((1, h, d), lambda i, be, bv: (be[i], 0, 0)),
            pl.BlockSpec((1, d, h), lambda i, be, bv: (be[i], 0, 0)),
            pl.BlockSpec((BLK, 1), lambda i, be, bv: (i, 0)),
        ],
        out_specs=pl.BlockSpec((BLK, d // 2), lambda i, be, bv: (i, 0)),
    )
    wout_pk = pl.pallas_call(
        _mlp_body,
        grid_spec=grid_spec,
        out_shape=jax.ShapeDtypeStruct((npad, d // 2), jnp.int32),
    )(block_expert, block_valid, xs, Wg, Wu, Wd, row_w.reshape(npad, 1))

    final = _sc_combine(wout_pk, pos0, pos1, d)

    return final.reshape(b, s, d), logits
